# baseline (speedup 1.0000x reference)
import jax
import jax.numpy as jnp
from jax.experimental import pallas as pl
from jax.experimental.pallas import tpu as pltpu

_BM = 1024


def _fused_heads_t(x_ref, wc_ref, bc_ref, wb_ref, bb_ref, sc_ref, pd_ref):
    xt = x_ref[...]
    dims = (((1,), (1,)), ((), ()))
    sc_ref[...] = (
        jax.lax.dot_general(wc_ref[...], xt, dims, preferred_element_type=jnp.float32)
        + bc_ref[...]
    )
    pd_ref[...] = (
        jax.lax.dot_general(wb_ref[...], xt, dims, preferred_element_type=jnp.float32)
        + bb_ref[...]
    )


def kernel(x, Wc, bc, Wb, bb):
    if x.ndim > 2:
        x = x.reshape(x.shape[0], -1)
    n, k = x.shape
    nc = Wc.shape[0]
    nb = Wb.shape[0]
    scores_t, deltas_t = pl.pallas_call(
        _fused_heads_t,
        grid=(pl.cdiv(n, _BM),),
        in_specs=[
            pl.BlockSpec((_BM, k), lambda i: (i, 0)),
            pl.BlockSpec((nc, k), lambda i: (0, 0)),
            pl.BlockSpec((nc, 1), lambda i: (0, 0)),
            pl.BlockSpec((nb, k), lambda i: (0, 0)),
            pl.BlockSpec((nb, 1), lambda i: (0, 0)),
        ],
        out_specs=[
            pl.BlockSpec((nc, _BM), lambda i: (0, i)),
            pl.BlockSpec((nb, _BM), lambda i: (0, i)),
        ],
        out_shape=[
            jax.ShapeDtypeStruct((nc, n), x.dtype),
            jax.ShapeDtypeStruct((nb, n), x.dtype),
        ],
        compiler_params=pltpu.CompilerParams(
            dimension_semantics=("parallel",),
        ),
    )(x, Wc, bc.reshape(nc, 1), Wb, bb.reshape(nb, 1))
    return (scores_t.T, deltas_t.T)


# speedup vs baseline: 1.0999x; 1.0999x over previous
import jax
import jax.numpy as jnp
from jax.experimental import pallas as pl
from jax.experimental.pallas import tpu as pltpu

_BM = 4096


def _fused_heads_t(x_ref, wc_ref, bc_ref, wb_ref, bb_ref, sc_ref, pd_ref):
    xt = x_ref[...]
    dims = (((1,), (1,)), ((), ()))
    sc_ref[...] = (
        jax.lax.dot_general(wc_ref[...], xt, dims, preferred_element_type=jnp.float32)
        + bc_ref[...]
    )
    pd_ref[...] = (
        jax.lax.dot_general(wb_ref[...], xt, dims, preferred_element_type=jnp.float32)
        + bb_ref[...]
    )


def kernel(x, Wc, bc, Wb, bb):
    if x.ndim > 2:
        x = x.reshape(x.shape[0], -1)
    n, k = x.shape
    nc = Wc.shape[0]
    nb = Wb.shape[0]
    scores_t, deltas_t = pl.pallas_call(
        _fused_heads_t,
        grid=(pl.cdiv(n, _BM),),
        in_specs=[
            pl.BlockSpec((_BM, k), lambda i: (i, 0)),
            pl.BlockSpec((nc, k), lambda i: (0, 0)),
            pl.BlockSpec((nc, 1), lambda i: (0, 0)),
            pl.BlockSpec((nb, k), lambda i: (0, 0)),
            pl.BlockSpec((nb, 1), lambda i: (0, 0)),
        ],
        out_specs=[
            pl.BlockSpec((nc, _BM), lambda i: (0, i)),
            pl.BlockSpec((nb, _BM), lambda i: (0, i)),
        ],
        out_shape=[
            jax.ShapeDtypeStruct((nc, n), x.dtype),
            jax.ShapeDtypeStruct((nb, n), x.dtype),
        ],
        compiler_params=pltpu.CompilerParams(
            dimension_semantics=("parallel",),
        ),
    )(x, Wc, bc.reshape(nc, 1), Wb, bb.reshape(nb, 1))
    return (scores_t.T, deltas_t.T)


# speedup vs baseline: 1.1303x; 1.0276x over previous
import jax
import jax.numpy as jnp
from jax.experimental import pallas as pl
from jax.experimental.pallas import tpu as pltpu

_BM = 2560


def _fused_heads_t(x_ref, wc_ref, bc_ref, wb_ref, bb_ref, sc_ref, pd_ref):
    xt = x_ref[...]
    dims = (((1,), (1,)), ((), ()))
    sc_ref[...] = (
        jax.lax.dot_general(wc_ref[...], xt, dims, preferred_element_type=jnp.float32)
        + bc_ref[...]
    )
    pd_ref[...] = (
        jax.lax.dot_general(wb_ref[...], xt, dims, preferred_element_type=jnp.float32)
        + bb_ref[...]
    )


def kernel(x, Wc, bc, Wb, bb):
    if x.ndim > 2:
        x = x.reshape(x.shape[0], -1)
    n, k = x.shape
    nc = Wc.shape[0]
    nb = Wb.shape[0]
    scores_t, deltas_t = pl.pallas_call(
        _fused_heads_t,
        grid=(pl.cdiv(n, _BM),),
        in_specs=[
            pl.BlockSpec((_BM, k), lambda i: (i, 0)),
            pl.BlockSpec((nc, k), lambda i: (0, 0)),
            pl.BlockSpec((nc, 1), lambda i: (0, 0)),
            pl.BlockSpec((nb, k), lambda i: (0, 0)),
            pl.BlockSpec((nb, 1), lambda i: (0, 0)),
        ],
        out_specs=[
            pl.BlockSpec((nc, _BM), lambda i: (0, i)),
            pl.BlockSpec((nb, _BM), lambda i: (0, i)),
        ],
        out_shape=[
            jax.ShapeDtypeStruct((nc, n), x.dtype),
            jax.ShapeDtypeStruct((nb, n), x.dtype),
        ],
        compiler_params=pltpu.CompilerParams(
            dimension_semantics=("parallel",),
        ),
    )(x, Wc, bc.reshape(nc, 1), Wb, bb.reshape(nb, 1))
    return (scores_t.T, deltas_t.T)
